# initial kernel scaffold (unmeasured)
import jax
import jax.numpy as jnp
from jax import lax
from jax.experimental import pallas as pl
from jax.experimental.pallas import tpu as pltpu

N_DEV = 8
B_LOC = 2
B = 16
SQ = 128
D = 512
HQ_LOC = 4
DH = 64
HD_LOC = HQ_LOC * DH


def _rope_tables():
    lane = lax.broadcasted_iota(jnp.int32, (SQ, HD_LOC), 1)
    pos = lax.broadcasted_iota(jnp.float32, (SQ, HD_LOC), 0)
    k = (lane % DH) // 2
    inv = jnp.exp(k.astype(jnp.float32) * (-2.0 / DH * jnp.log(10000.0)))
    ang = pos * inv
    return jnp.cos(ang), jnp.sin(ang)


def _rotate_half(t2):
    r_up = pltpu.roll(t2, 1, 1)
    r_dn = pltpu.roll(t2, -1, 1)
    lane = lax.broadcasted_iota(jnp.int32, t2.shape, 1)
    return jnp.where(lane % 2 == 0, -r_dn, r_up)


def kernel(x, Wq, Wk, Wv, Wo):
    def body(x_ref, wq_ref, wk_ref, wv_ref, wo_ref, out_ref,
             xg, qs, ks, vs, cs, partial, stg,
             ag_send, ag_recv, rs_send, rs_recv):
        my = lax.axis_index("i")
        left = (my + N_DEV - 1) % N_DEV
        right = (my + 1) % N_DEV

        barrier = pltpu.get_barrier_semaphore()
        for nbr in (left, right):
            pl.semaphore_signal(
                barrier, inc=1, device_id=(nbr,),
                device_id_type=pl.DeviceIdType.MESH,
            )
        pl.semaphore_wait(barrier, 2)

        xg[my] = x_ref[...].astype(jnp.bfloat16)

        for s in range(N_DEV - 1):
            c_send = (my - s) % N_DEV
            rdma = pltpu.make_async_remote_copy(
                src_ref=xg.at[c_send],
                dst_ref=xg.at[c_send],
                send_sem=ag_send.at[s],
                recv_sem=ag_recv.at[s],
                device_id=(right,),
                device_id_type=pl.DeviceIdType.MESH,
            )
            rdma.start()
            rdma.wait()

        x2 = xg[...].reshape(B * SQ, D)
        wq = wq_ref[...].astype(jnp.bfloat16)
        wk = wk_ref[...].astype(jnp.bfloat16)
        wv = wv_ref[...].astype(jnp.bfloat16)

        cos, sin = _rope_tables()
        cos3 = cos[None, :, :]
        sin3 = sin[None, :, :]

        def proj_rope(w):
            t2 = jnp.dot(x2, w, preferred_element_type=jnp.float32)
            tr2 = _rotate_half(t2)
            t3 = t2.reshape(B, SQ, HD_LOC)
            tr3 = tr2.reshape(B, SQ, HD_LOC)
            return t3 * cos3 + tr3 * sin3

        qs[...] = proj_rope(wq).astype(jnp.bfloat16)
        ks[...] = proj_rope(wk).astype(jnp.bfloat16)
        vs[...] = (
            jnp.dot(x2, wv, preferred_element_type=jnp.float32)
            .reshape(B, SQ, HD_LOC)
            .astype(jnp.bfloat16)
        )

        def attn_batch(g, carry):
            qb = qs[pl.ds(g, 1)].reshape(SQ, HD_LOC)
            kb = ks[pl.ds(g, 1)].reshape(SQ, HD_LOC)
            vb = vs[pl.ds(g, 1)].reshape(SQ, HD_LOC)
            for h in range(HQ_LOC):
                sl = slice(h * DH, (h + 1) * DH)
                sc = lax.dot_general(
                    qb[:, sl], kb[:, sl],
                    (((1,), (1,)), ((), ())),
                    preferred_element_type=jnp.float32,
                ) * 0.125
                m = jnp.max(sc, axis=-1, keepdims=True)
                w = jnp.exp(sc - m)
                w = w / jnp.sum(w, axis=-1, keepdims=True)
                ctx = jnp.dot(
                    w.astype(jnp.bfloat16), vb[:, sl],
                    preferred_element_type=jnp.float32,
                )
                cs[pl.ds(g, 1), :, sl] = ctx[None].astype(jnp.bfloat16)
            return carry

        lax.fori_loop(0, B, attn_batch, 0)

        ctx2 = cs[...].reshape(B * SQ, HD_LOC)
        wo = wo_ref[...].astype(jnp.bfloat16)
        part = jnp.dot(ctx2, wo, preferred_element_type=jnp.float32)
        partial[...] = part.reshape(N_DEV, B_LOC, SQ, D)

        for s in range(N_DEV - 1):
            c_send = (my - 1 - s) % N_DEV
            rdma = pltpu.make_async_remote_copy(
                src_ref=partial.at[c_send],
                dst_ref=stg.at[s],
                send_sem=rs_send.at[s],
                recv_sem=rs_recv.at[s],
                device_id=(right,),
                device_id_type=pl.DeviceIdType.MESH,
            )
            rdma.start()
            rdma.wait()
            c_recv = (my - 2 - s) % N_DEV
            partial[c_recv] = partial[c_recv] + stg[s]

        out_ref[...] = partial[pl.ds(my, 1)].reshape(B_LOC, SQ, D)

    out_shape = jax.ShapeDtypeStruct((B_LOC, SQ, D), jnp.float32)
    return pl.pallas_call(
        body,
        out_shape=out_shape,
        in_specs=[pl.BlockSpec(memory_space=pltpu.VMEM)] * 5,
        out_specs=pl.BlockSpec(memory_space=pltpu.VMEM),
        scratch_shapes=[
            pltpu.VMEM((N_DEV, B_LOC, SQ, D), jnp.bfloat16),
            pltpu.VMEM((B, SQ, HD_LOC), jnp.bfloat16),
            pltpu.VMEM((B, SQ, HD_LOC), jnp.bfloat16),
            pltpu.VMEM((B, SQ, HD_LOC), jnp.bfloat16),
            pltpu.VMEM((B, SQ, HD_LOC), jnp.bfloat16),
            pltpu.VMEM((N_DEV, B_LOC, SQ, D), jnp.float32),
            pltpu.VMEM((N_DEV - 1, B_LOC, SQ, D), jnp.float32),
            pltpu.SemaphoreType.DMA((N_DEV - 1,)),
            pltpu.SemaphoreType.DMA((N_DEV - 1,)),
            pltpu.SemaphoreType.DMA((N_DEV - 1,)),
            pltpu.SemaphoreType.DMA((N_DEV - 1,)),
        ],
        compiler_params=pltpu.CompilerParams(collective_id=0),
    )(x, Wq, Wk, Wv, Wo)


# baseline (device time: 122901 ns/iter reference)
import jax
import jax.numpy as jnp
from jax import lax
from jax.experimental import pallas as pl
from jax.experimental.pallas import tpu as pltpu

N_DEV = 8
B_LOC = 2
B = 16
SQ = 128
D = 512
HQ_LOC = 4
DH = 64
HD_LOC = HQ_LOC * DH


def _rope_tables():
    lane = lax.broadcasted_iota(jnp.int32, (SQ, HD_LOC), 1)
    pos = lax.broadcasted_iota(jnp.int32, (SQ, HD_LOC), 0).astype(jnp.float32)
    k = (lane % DH) // 2
    inv = jnp.exp(k.astype(jnp.float32) * (-2.0 / DH * jnp.log(10000.0)))
    ang = pos * inv
    return jnp.cos(ang), jnp.sin(ang)


def _rotate_half(t2):
    r_up = pltpu.roll(t2, 1, 1)
    r_dn = pltpu.roll(t2, t2.shape[1] - 1, 1)
    lane = lax.broadcasted_iota(jnp.int32, t2.shape, 1)
    return jnp.where(lane % 2 == 0, -r_dn, r_up)


def kernel(x, Wq, Wk, Wv, Wo):
    def body(x_ref, wq_ref, wk_ref, wv_ref, wo_ref, out_ref,
             xg, qs, ks, vs, cs, partial, stg,
             ag_send, ag_recv, rs_send, rs_recv):
        my = lax.axis_index("i")
        left = (my + N_DEV - 1) % N_DEV
        right = (my + 1) % N_DEV

        barrier = pltpu.get_barrier_semaphore()
        for nbr in (left, right):
            pl.semaphore_signal(
                barrier, inc=1, device_id=(nbr,),
                device_id_type=pl.DeviceIdType.MESH,
            )
        pl.semaphore_wait(barrier, 2)

        xg[my] = x_ref[...].astype(jnp.bfloat16)

        for s in range(N_DEV - 1):
            c_send = (my - s) % N_DEV
            rdma = pltpu.make_async_remote_copy(
                src_ref=xg.at[c_send],
                dst_ref=xg.at[c_send],
                send_sem=ag_send.at[s],
                recv_sem=ag_recv.at[s],
                device_id=(right,),
                device_id_type=pl.DeviceIdType.MESH,
            )
            rdma.start()
            rdma.wait()

        x2 = xg[...].reshape(B * SQ, D)
        wq = wq_ref[...].astype(jnp.bfloat16)
        wk = wk_ref[...].astype(jnp.bfloat16)
        wv = wv_ref[...].astype(jnp.bfloat16)

        cos, sin = _rope_tables()
        cos3 = cos[None, :, :]
        sin3 = sin[None, :, :]

        def proj_rope(w):
            t2 = jnp.dot(x2, w, preferred_element_type=jnp.float32)
            tr2 = _rotate_half(t2)
            t3 = t2.reshape(B, SQ, HD_LOC)
            tr3 = tr2.reshape(B, SQ, HD_LOC)
            return t3 * cos3 + tr3 * sin3

        qs[...] = proj_rope(wq).astype(jnp.bfloat16)
        ks[...] = proj_rope(wk).astype(jnp.bfloat16)
        vs[...] = (
            jnp.dot(x2, wv, preferred_element_type=jnp.float32)
            .reshape(B, SQ, HD_LOC)
            .astype(jnp.bfloat16)
        )

        def attn_batch(g, carry):
            qb = qs[pl.ds(g, 1)].reshape(SQ, HD_LOC)
            kb = ks[pl.ds(g, 1)].reshape(SQ, HD_LOC)
            vb = vs[pl.ds(g, 1)].reshape(SQ, HD_LOC)
            for h in range(HQ_LOC):
                sl = slice(h * DH, (h + 1) * DH)
                sc = lax.dot_general(
                    qb[:, sl], kb[:, sl],
                    (((1,), (1,)), ((), ())),
                    preferred_element_type=jnp.float32,
                ) * 0.125
                m = jnp.max(sc, axis=-1, keepdims=True)
                w = jnp.exp(sc - m)
                w = w / jnp.sum(w, axis=-1, keepdims=True)
                ctx = jnp.dot(
                    w.astype(jnp.bfloat16), vb[:, sl],
                    preferred_element_type=jnp.float32,
                )
                cs[pl.ds(g, 1), :, sl] = ctx[None].astype(jnp.bfloat16)
            return carry

        lax.fori_loop(0, B, attn_batch, 0)

        ctx2 = cs[...].reshape(B * SQ, HD_LOC)
        wo = wo_ref[...].astype(jnp.bfloat16)
        part = jnp.dot(ctx2, wo, preferred_element_type=jnp.float32)
        partial[...] = part.reshape(N_DEV, B_LOC, SQ, D)

        for s in range(N_DEV - 1):
            c_send = (my - 1 - s) % N_DEV
            rdma = pltpu.make_async_remote_copy(
                src_ref=partial.at[c_send],
                dst_ref=stg.at[s],
                send_sem=rs_send.at[s],
                recv_sem=rs_recv.at[s],
                device_id=(right,),
                device_id_type=pl.DeviceIdType.MESH,
            )
            rdma.start()
            rdma.wait()
            c_recv = (my - 2 - s) % N_DEV
            partial[c_recv] = partial[c_recv] + stg[s]

        out_ref[...] = partial[pl.ds(my, 1)].reshape(B_LOC, SQ, D)

    out_shape = jax.ShapeDtypeStruct((B_LOC, SQ, D), jnp.float32)
    return pl.pallas_call(
        body,
        out_shape=out_shape,
        in_specs=[pl.BlockSpec(memory_space=pltpu.VMEM)] * 5,
        out_specs=pl.BlockSpec(memory_space=pltpu.VMEM),
        scratch_shapes=[
            pltpu.VMEM((N_DEV, B_LOC, SQ, D), jnp.bfloat16),
            pltpu.VMEM((B, SQ, HD_LOC), jnp.bfloat16),
            pltpu.VMEM((B, SQ, HD_LOC), jnp.bfloat16),
            pltpu.VMEM((B, SQ, HD_LOC), jnp.bfloat16),
            pltpu.VMEM((B, SQ, HD_LOC), jnp.bfloat16),
            pltpu.VMEM((N_DEV, B_LOC, SQ, D), jnp.float32),
            pltpu.VMEM((N_DEV - 1, B_LOC, SQ, D), jnp.float32),
            pltpu.SemaphoreType.DMA((N_DEV - 1,)),
            pltpu.SemaphoreType.DMA((N_DEV - 1,)),
            pltpu.SemaphoreType.DMA((N_DEV - 1,)),
            pltpu.SemaphoreType.DMA((N_DEV - 1,)),
        ],
        compiler_params=pltpu.CompilerParams(collective_id=0),
    )(x, Wq, Wk, Wv, Wo)


# device time: 55883 ns/iter; 2.1993x vs baseline; 2.1993x over previous
import jax
import jax.numpy as jnp
from jax import lax
from jax.experimental import pallas as pl
from jax.experimental.pallas import tpu as pltpu

N_DEV = 8
B_LOC = 2
B = 16
SQ = 128
D = 512
HQ_LOC = 4
DH = 64
HD_LOC = HQ_LOC * DH
R_LOC = B_LOC * SQ


def _rope_tables():
    lane = lax.broadcasted_iota(jnp.int32, (SQ, HD_LOC), 1)
    pos = lax.broadcasted_iota(jnp.int32, (SQ, HD_LOC), 0).astype(jnp.float32)
    k = (lane % DH) // 2
    inv = jnp.exp(k.astype(jnp.float32) * (-2.0 / DH * jnp.log(10000.0)))
    ang = pos * inv
    return jnp.cos(ang), jnp.sin(ang)


def _rotate_half(t2):
    r_up = pltpu.roll(t2, 1, 1)
    r_dn = pltpu.roll(t2, t2.shape[1] - 1, 1)
    lane = lax.broadcasted_iota(jnp.int32, t2.shape, 1)
    return jnp.where(lane % 2 == 0, -r_dn, r_up)


def kernel(x, Wq, Wk, Wv, Wo):
    def body(x_ref, wq_ref, wk_ref, wv_ref, wo_ref, out_ref,
             xg, partial, stg, ag_send, ag_recv, rs_send, rs_recv):
        my = lax.axis_index("i")
        left = (my + N_DEV - 1) % N_DEV
        right = (my + 1) % N_DEV

        barrier = pltpu.get_barrier_semaphore()
        for nbr in (left, right):
            pl.semaphore_signal(
                barrier, inc=1, device_id=(nbr,),
                device_id_type=pl.DeviceIdType.MESH,
            )
        pl.semaphore_wait(barrier, 2)

        wq = wq_ref[...].astype(jnp.bfloat16)
        wk = wk_ref[...].astype(jnp.bfloat16)
        wv = wv_ref[...].astype(jnp.bfloat16)
        wo = wo_ref[...].astype(jnp.bfloat16)
        cos, sin = _rope_tables()
        cos3, sin3 = cos[None], sin[None]

        def chunk_partial(xc):
            def proj_rope(w):
                t2 = jnp.dot(xc, w, preferred_element_type=jnp.float32)
                tr2 = _rotate_half(t2)
                t3 = t2.reshape(B_LOC, SQ, HD_LOC)
                tr3 = tr2.reshape(B_LOC, SQ, HD_LOC)
                return (t3 * cos3 + tr3 * sin3).astype(jnp.bfloat16)

            q3 = proj_rope(wq)
            k3 = proj_rope(wk)
            v3 = (
                jnp.dot(xc, wv, preferred_element_type=jnp.float32)
                .reshape(B_LOC, SQ, HD_LOC)
                .astype(jnp.bfloat16)
            )
            rows = []
            for b in range(B_LOC):
                ctxs = []
                for h in range(HQ_LOC):
                    sl = slice(h * DH, (h + 1) * DH)
                    sc = lax.dot_general(
                        q3[b, :, sl], k3[b, :, sl],
                        (((1,), (1,)), ((), ())),
                        preferred_element_type=jnp.float32,
                    ) * 0.125
                    m = jnp.max(sc, axis=-1, keepdims=True)
                    w = jnp.exp(sc - m)
                    w = w / jnp.sum(w, axis=-1, keepdims=True)
                    ctxs.append(jnp.dot(
                        w.astype(jnp.bfloat16), v3[b, :, sl],
                        preferred_element_type=jnp.float32,
                    ))
                rows.append(jnp.concatenate(ctxs, axis=1))
            ctx2 = jnp.stack(rows).reshape(R_LOC, HD_LOC).astype(jnp.bfloat16)
            part = jnp.dot(ctx2, wo, preferred_element_type=jnp.float32)
            return part.reshape(B_LOC, SQ, D)

        def ag_hop(s):
            c = (my - s) % N_DEV
            rdma = pltpu.make_async_remote_copy(
                src_ref=xg.at[c], dst_ref=xg.at[c],
                send_sem=ag_send.at[s], recv_sem=ag_recv.at[s],
                device_id=(right,), device_id_type=pl.DeviceIdType.MESH,
            )
            rdma.start()
            return rdma

        xg[my] = x_ref[...].astype(jnp.bfloat16)
        ag = [ag_hop(0)]
        partial[my] = chunk_partial(
            x_ref[...].reshape(R_LOC, D).astype(jnp.bfloat16)
        ).astype(jnp.bfloat16)

        rs = []
        for s in range(N_DEV - 1):
            ag[s].wait_recv()
            if s < N_DEV - 2:
                ag.append(ag_hop(s + 1))
            c = (my - 1 - s) % N_DEV
            part = chunk_partial(xg[pl.ds(c, 1)].reshape(R_LOC, D))
            if s > 0:
                rs[s - 1].wait_recv()
                part = part + stg[s - 1].astype(jnp.float32)
            partial[c] = part.astype(jnp.bfloat16)
            rdma = pltpu.make_async_remote_copy(
                src_ref=partial.at[c], dst_ref=stg.at[s],
                send_sem=rs_send.at[s], recv_sem=rs_recv.at[s],
                device_id=(right,), device_id_type=pl.DeviceIdType.MESH,
            )
            rdma.start()
            rs.append(rdma)

        rs[N_DEV - 2].wait_recv()
        out_ref[...] = (
            partial[pl.ds(my, 1)].reshape(B_LOC, SQ, D).astype(jnp.float32)
            + stg[N_DEV - 2].astype(jnp.float32)
        )

        for r in ag + rs:
            r.wait_send()

    out_shape = jax.ShapeDtypeStruct((B_LOC, SQ, D), jnp.float32)
    return pl.pallas_call(
        body,
        out_shape=out_shape,
        in_specs=[pl.BlockSpec(memory_space=pltpu.VMEM)] * 5,
        out_specs=pl.BlockSpec(memory_space=pltpu.VMEM),
        scratch_shapes=[
            pltpu.VMEM((N_DEV, B_LOC, SQ, D), jnp.bfloat16),
            pltpu.VMEM((N_DEV, B_LOC, SQ, D), jnp.bfloat16),
            pltpu.VMEM((N_DEV - 1, B_LOC, SQ, D), jnp.bfloat16),
            pltpu.SemaphoreType.DMA((N_DEV - 1,)),
            pltpu.SemaphoreType.DMA((N_DEV - 1,)),
            pltpu.SemaphoreType.DMA((N_DEV - 1,)),
            pltpu.SemaphoreType.DMA((N_DEV - 1,)),
        ],
        compiler_params=pltpu.CompilerParams(collective_id=0),
    )(x, Wq, Wk, Wv, Wo)
